# Initial kernel scaffold; baseline (speedup 1.0000x reference)
import jax
import jax.numpy as jnp
from jax.experimental import pallas as pl


def kernel(features, projection, origin):
    raise NotImplementedError("write your pallas kernel here")


# speedup vs baseline: 512.2340x; 512.2340x over previous
import functools

import numpy as np
import jax
import jax.numpy as jnp
from jax import lax
from jax.experimental import pallas as pl
from jax.experimental.pallas import tpu as pltpu
from jax.experimental.pallas import tpu_sc as plsc

NX, NY, NZ = 128, 128, 64
N = NX * NY * NZ
VOX = 0.04

ROWS, LANES, RB = 8192, 128, 512
MAGIC = np.float32(12582912.0)

NC, NS, L = 2, 16, 16
NW = NC * NS

CPG = 4
GROUPS = 8
SEGS = NW // GROUPS
SEG = N // SEGS

HW_PAD = 19216
SENTINEL = 19200

CHUNK = 8192
VECS = CHUNK // L
NCHUNKS = SEG // CHUNK


def _tc_index(W, H, params_ref, linm_ref, valid_ref):
    i = pl.program_id(0)
    r = lax.broadcasted_iota(jnp.int32, (RB, LANES), 0)
    lcol = lax.broadcasted_iota(jnp.int32, (RB, LANES), 1)
    n = (i * RB + r) * LANES + lcol
    iz = n & (NZ - 1)
    iy = (n >> 6) & (NY - 1)
    ix = n >> 13
    ox, oy, oz = params_ref[12], params_ref[13], params_ref[14]
    wx = ix.astype(jnp.float32) * VOX + ox
    wy = iy.astype(jnp.float32) * VOX + oy
    wz = iz.astype(jnp.float32) * VOX + oz
    wxb = wx.astype(jnp.bfloat16).astype(jnp.float32)
    wyb = wy.astype(jnp.bfloat16).astype(jnp.float32)
    wzb = wz.astype(jnp.bfloat16).astype(jnp.float32)
    p = [params_ref[j] for j in range(12)]
    cx = ((p[0] * wxb + p[1] * wyb) + p[2] * wzb) + p[3]
    cy = ((p[4] * wxb + p[5] * wyb) + p[6] * wzb) + p[7]
    cz = ((p[8] * wxb + p[9] * wyb) + p[10] * wzb) + p[11]
    qx = cx / cz
    qy = cy / cz
    xr = (qx + MAGIC) - MAGIC
    yr = (qy + MAGIC) - MAGIC
    xi = xr.astype(jnp.int32)
    yi = yr.astype(jnp.int32)
    valid = ((xi >= 0) & (yi >= 0) & (xi < W) & (yi < H)
             & (cz > jnp.float32(0.0)))
    xc = jnp.minimum(jnp.maximum(xi, 0), W - 1)
    yc = jnp.minimum(jnp.maximum(yi, 0), H - 1)
    lin = yc * W + xc
    linm_ref[...] = jnp.where(valid, lin, SENTINEL)
    valid_ref[...] = valid.astype(jnp.int32)


def _sc_gather(table_hbm, linm_hbm, vol_hbm,
               t0_v, t1_v, t2_v, t3_v, idx_v, out_v):
    wid = lax.axis_index("s") * NC + lax.axis_index("c")
    group = wid % GROUPS
    seg = wid // GROUPS
    tables = [t0_v, t1_v, t2_v, t3_v]

    for j in range(CPG):
        c = group * CPG + j
        pltpu.sync_copy(table_hbm.at[pl.ds(c * HW_PAD, HW_PAD)], tables[j])

    seg_base = seg * SEG

    def chunk_body(ci, _):
        base = seg_base + ci * CHUNK
        pltpu.sync_copy(linm_hbm.at[pl.ds(base, CHUNK)], idx_v)

        def vec_body(vi, _):
            sl = pl.ds(vi * L, L)
            linm = idx_v[sl]
            for j in range(CPG):
                out_v[j, sl] = plsc.load_gather(tables[j], [linm])
            return 0

        lax.fori_loop(0, VECS, vec_body, 0)

        for j in range(CPG):
            c = group * CPG + j
            pltpu.sync_copy(out_v.at[j],
                            vol_hbm.at[pl.ds(c * N + base, CHUNK)])
        return 0

    lax.fori_loop(0, NCHUNKS, chunk_body, 0)


def kernel(features, projection, origin):
    B, C, H, W = features.shape

    pb = projection.reshape(-1).astype(jnp.bfloat16).astype(jnp.float32)
    params = jnp.concatenate([
        pb, origin.reshape(-1).astype(jnp.float32),
        jnp.zeros((1,), jnp.float32)])

    linm, valid = pl.pallas_call(
        functools.partial(_tc_index, W, H),
        grid=(ROWS // RB,),
        in_specs=[pl.BlockSpec(memory_space=pltpu.SMEM)],
        out_specs=[pl.BlockSpec((RB, LANES), lambda i: (i, 0)),
                   pl.BlockSpec((RB, LANES), lambda i: (i, 0))],
        out_shape=[jax.ShapeDtypeStruct((ROWS, LANES), jnp.int32),
                   jax.ShapeDtypeStruct((ROWS, LANES), jnp.int32)],
    )(params)

    table = jnp.zeros((C, HW_PAD), features.dtype)
    table = table.at[:, :H * W].set(features.reshape(C, H * W))
    table = table.reshape(C * HW_PAD)

    mesh = plsc.VectorSubcoreMesh(core_axis_name="c", subcore_axis_name="s")
    vol_flat = pl.kernel(
        _sc_gather,
        out_type=jax.ShapeDtypeStruct((C * N,), jnp.float32),
        mesh=mesh,
        compiler_params=pltpu.CompilerParams(needs_layout_passes=False),
        scratch_types=[
            pltpu.VMEM((HW_PAD,), jnp.float32),
            pltpu.VMEM((HW_PAD,), jnp.float32),
            pltpu.VMEM((HW_PAD,), jnp.float32),
            pltpu.VMEM((HW_PAD,), jnp.float32),
            pltpu.VMEM((CHUNK,), jnp.int32),
            pltpu.VMEM((CPG, CHUNK), jnp.float32),
        ],
    )(table, linm.reshape(-1))

    volume = vol_flat.reshape(B, C, NX, NY, NZ)
    valid_vol = valid.reshape(-1).astype(bool).reshape(B, 1, NX, NY, NZ)
    return volume, valid_vol


# speedup vs baseline: 571.7501x; 1.1162x over previous
import functools

import numpy as np
import jax
import jax.numpy as jnp
from jax import lax
from jax.experimental import pallas as pl
from jax.experimental.pallas import tpu as pltpu
from jax.experimental.pallas import tpu_sc as plsc

NX, NY, NZ = 128, 128, 64
N = NX * NY * NZ
VOX = 0.04

ROWS, LANES, RB = 8192, 128, 512
MAGIC = np.float32(12582912.0)

NC, NS, L = 2, 16, 16
NW = NC * NS

CPG = 4
GROUPS = 8
SEGS = NW // GROUPS
SEG = N // SEGS

HW_IMG = 19200
HW_PAD = 19216
SENTINEL = 19200

CHUNK = 4096
VECS = CHUNK // L
UNROLL = 4
NCHUNKS = SEG // CHUNK


def _tc_index(W, H, params_ref, linm_ref, valid_ref):
    i = pl.program_id(0)
    r = lax.broadcasted_iota(jnp.int32, (RB, LANES), 0)
    lcol = lax.broadcasted_iota(jnp.int32, (RB, LANES), 1)
    n = (i * RB + r) * LANES + lcol
    iz = n & (NZ - 1)
    iy = (n >> 6) & (NY - 1)
    ix = n >> 13
    ox, oy, oz = params_ref[12], params_ref[13], params_ref[14]
    wx = ix.astype(jnp.float32) * VOX + ox
    wy = iy.astype(jnp.float32) * VOX + oy
    wz = iz.astype(jnp.float32) * VOX + oz
    wxb = wx.astype(jnp.bfloat16).astype(jnp.float32)
    wyb = wy.astype(jnp.bfloat16).astype(jnp.float32)
    wzb = wz.astype(jnp.bfloat16).astype(jnp.float32)
    p = [params_ref[j] for j in range(12)]
    cx = ((p[0] * wxb + p[1] * wyb) + p[2] * wzb) + p[3]
    cy = ((p[4] * wxb + p[5] * wyb) + p[6] * wzb) + p[7]
    cz = ((p[8] * wxb + p[9] * wyb) + p[10] * wzb) + p[11]
    qx = cx / cz
    qy = cy / cz
    xr = (qx + MAGIC) - MAGIC
    yr = (qy + MAGIC) - MAGIC
    xi = xr.astype(jnp.int32)
    yi = yr.astype(jnp.int32)
    valid = ((xi >= 0) & (yi >= 0) & (xi < W) & (yi < H)
             & (cz > jnp.float32(0.0)))
    xc = jnp.minimum(jnp.maximum(xi, 0), W - 1)
    yc = jnp.minimum(jnp.maximum(yi, 0), H - 1)
    lin = yc * W + xc
    linm_ref[...] = jnp.where(valid, lin, SENTINEL)
    valid_ref[...] = valid


def _sc_gather(feat_hbm, linm_hbm, vol_hbm,
               t0_v, t1_v, t2_v, t3_v, idx_a, idx_b, out_a, out_b,
               sem_i, sem_oa, sem_ob):
    wid = lax.axis_index("s") * NC + lax.axis_index("c")
    group = wid % GROUPS
    seg = wid // GROUPS
    tables = [t0_v, t1_v, t2_v, t3_v]
    idx_bufs = (idx_a, idx_b)
    out_bufs = (out_a, out_b)
    out_sems = (sem_oa, sem_ob)

    for j in range(CPG):
        c = group * CPG + j
        pltpu.sync_copy(feat_hbm.at[pl.ds(c * HW_IMG, HW_IMG)],
                        tables[j].at[pl.ds(0, HW_IMG)])
        tables[j][pl.ds(HW_IMG, HW_PAD - HW_IMG)] = jnp.zeros(
            (HW_PAD - HW_IMG,), jnp.float32)

    seg_base = seg * SEG
    pltpu.async_copy(linm_hbm.at[pl.ds(seg_base, CHUNK)], idx_a, sem_i)

    def pair_body(ci0, _):
        for b in (0, 1):
            ci = ci0 * 2 + b
            base = seg_base + ci * CHUNK
            ib, ob, so = idx_bufs[b], out_bufs[b], out_sems[b]
            pltpu.make_async_copy(
                linm_hbm.at[pl.ds(base, CHUNK)], ib, sem_i).wait()

            @pl.when(ci < NCHUNKS - 1)
            def _():
                pltpu.async_copy(
                    linm_hbm.at[pl.ds(base + CHUNK, CHUNK)],
                    idx_bufs[1 - b], sem_i)

            @pl.when(ci >= 2)
            def _():
                for j in range(CPG):
                    c = group * CPG + j
                    pltpu.make_async_copy(
                        ob.at[j],
                        vol_hbm.at[pl.ds(c * N + base - 2 * CHUNK, CHUNK)],
                        so).wait()

            def vec_body(vi, _):
                for u in range(UNROLL):
                    sl = pl.ds((vi * UNROLL + u) * L, L)
                    linv = ib[sl]
                    for j in range(CPG):
                        ob[j, sl] = plsc.load_gather(tables[j], [linv])
                return 0

            lax.fori_loop(0, VECS // UNROLL, vec_body, 0)

            for j in range(CPG):
                c = group * CPG + j
                pltpu.async_copy(
                    ob.at[j], vol_hbm.at[pl.ds(c * N + base, CHUNK)], so)
        return 0

    lax.fori_loop(0, NCHUNKS // 2, pair_body, 0)

    for ci in (NCHUNKS - 2, NCHUNKS - 1):
        b = ci % 2
        base = seg_base + ci * CHUNK
        for j in range(CPG):
            c = group * CPG + j
            pltpu.make_async_copy(
                out_bufs[b].at[j],
                vol_hbm.at[pl.ds(c * N + base, CHUNK)],
                out_sems[b]).wait()


def kernel(features, projection, origin):
    B, C, H, W = features.shape

    pb = projection.reshape(-1).astype(jnp.bfloat16).astype(jnp.float32)
    params = jnp.concatenate([
        pb, origin.reshape(-1).astype(jnp.float32),
        jnp.zeros((1,), jnp.float32)])

    linm, valid = pl.pallas_call(
        functools.partial(_tc_index, W, H),
        grid=(ROWS // RB,),
        in_specs=[pl.BlockSpec(memory_space=pltpu.SMEM)],
        out_specs=[pl.BlockSpec((RB, LANES), lambda i: (i, 0)),
                   pl.BlockSpec((RB, LANES), lambda i: (i, 0))],
        out_shape=[jax.ShapeDtypeStruct((ROWS, LANES), jnp.int32),
                   jax.ShapeDtypeStruct((ROWS, LANES), jnp.bool_)],
    )(params)

    mesh = plsc.VectorSubcoreMesh(core_axis_name="c", subcore_axis_name="s")
    vol_flat = pl.kernel(
        _sc_gather,
        out_type=jax.ShapeDtypeStruct((C * N,), jnp.float32),
        mesh=mesh,
        compiler_params=pltpu.CompilerParams(needs_layout_passes=False),
        scratch_types=[
            pltpu.VMEM((HW_PAD,), jnp.float32),
            pltpu.VMEM((HW_PAD,), jnp.float32),
            pltpu.VMEM((HW_PAD,), jnp.float32),
            pltpu.VMEM((HW_PAD,), jnp.float32),
            pltpu.VMEM((CHUNK,), jnp.int32),
            pltpu.VMEM((CHUNK,), jnp.int32),
            pltpu.VMEM((CPG, CHUNK), jnp.float32),
            pltpu.VMEM((CPG, CHUNK), jnp.float32),
            pltpu.SemaphoreType.DMA,
            pltpu.SemaphoreType.DMA,
            pltpu.SemaphoreType.DMA,
        ],
    )(features.reshape(C * H * W), linm.reshape(-1))

    volume = vol_flat.reshape(B, C, NX, NY, NZ)
    valid_vol = valid.reshape(B, 1, NX, NY, NZ)
    return volume, valid_vol
